# speedup vs baseline: 1.2398x; 1.2398x over previous
import jax
import jax.numpy as jnp
from jax import lax
from jax.experimental import pallas as pl
from jax.experimental.pallas import tpu as pltpu
from jax.experimental.pallas import tpu_sc as plsc

NC = 2
NS = 16
NW = NC * NS

CHUNK = 64


def _gather_kernel(table_hbm, idx_hbm, out_hbm, idx_v, rows_v, gsem):
    wid = lax.axis_index("s") * NC + lax.axis_index("c")
    n_chunks = idx_hbm.shape[1]
    base = wid * (n_chunks * CHUNK)

    pltpu.sync_copy(idx_hbm.at[wid], idx_v)

    @pl.loop(0, n_chunks)
    def _(j):
        pltpu.async_copy(table_hbm.at[idx_v.at[j]], rows_v, gsem).wait()
        pltpu.sync_copy(rows_v, out_hbm.at[pl.ds(base + j * CHUNK, CHUNK)])


def kernel(input_ids, token_emb):
    B, T = input_ids.shape
    V, D = token_emb.shape
    n = B * T
    assert n % (NW * CHUNK) == 0
    n_chunks = n // (NW * CHUNK)

    idx = input_ids.reshape(NW, n_chunks, CHUNK).astype(jnp.int32)

    mesh = plsc.VectorSubcoreMesh(core_axis_name="c", subcore_axis_name="s")
    k = pl.kernel(
        _gather_kernel,
        out_type=jax.ShapeDtypeStruct((n, D), jnp.float32),
        mesh=mesh,
        scratch_types=[
            pltpu.VMEM((n_chunks, CHUNK), jnp.int32),
            pltpu.VMEM((CHUNK, D), jnp.float32),
            pltpu.SemaphoreType.DMA,
        ],
    )
    out = k(token_emb, idx)
    return out.reshape(B, T, D)


# speedup vs baseline: 1.2791x; 1.0317x over previous
import jax
import jax.numpy as jnp
from jax import lax
from jax.experimental import pallas as pl
from jax.experimental.pallas import tpu as pltpu
from jax.experimental.pallas import tpu_sc as plsc

NC = 2
NS = 16
NW = NC * NS

CHUNK = 64


def _gather_kernel(table_hbm, idx_hbm, out_hbm, idx_v, rows0, rows1,
                   g0, g1, w0, w1):
    wid = lax.axis_index("s") * NC + lax.axis_index("c")
    n_chunks = idx_hbm.shape[1]
    base = wid * (n_chunks * CHUNK)

    pltpu.sync_copy(idx_hbm.at[wid], idx_v)

    def gather_start(j, buf, sem):
        pltpu.async_copy(table_hbm.at[idx_v.at[j]], buf, sem)

    def gather_wait(buf, sem):
        pltpu.make_async_copy(table_hbm.at[idx_v.at[0]], buf, sem).wait()

    def write_start(j, buf, sem):
        pltpu.async_copy(buf, out_hbm.at[pl.ds(base + j * CHUNK, CHUNK)], sem)

    def write_wait(buf, sem):
        pltpu.make_async_copy(buf, out_hbm.at[pl.ds(base, CHUNK)], sem).wait()

    assert n_chunks % 2 == 1 and n_chunks >= 5
    gather_start(0, rows0, g0)
    gather_start(1, rows1, g1)

    @pl.loop(0, n_chunks - 3, step=2)
    def _(jj):
        gather_wait(rows0, g0)
        write_start(jj, rows0, w0)
        gather_wait(rows1, g1)
        write_start(jj + 1, rows1, w1)
        write_wait(rows0, w0)
        gather_start(jj + 2, rows0, g0)
        write_wait(rows1, w1)
        gather_start(jj + 3, rows1, g1)

    jj = n_chunks - 3
    gather_wait(rows0, g0)
    write_start(jj, rows0, w0)
    gather_wait(rows1, g1)
    write_start(jj + 1, rows1, w1)
    write_wait(rows0, w0)
    gather_start(jj + 2, rows0, g0)
    gather_wait(rows0, g0)
    write_start(jj + 2, rows0, w0)
    write_wait(rows1, w1)
    write_wait(rows0, w0)


def kernel(input_ids, token_emb):
    B, T = input_ids.shape
    V, D = token_emb.shape
    n = B * T
    assert n % (NW * CHUNK) == 0
    n_chunks = n // (NW * CHUNK)

    idx = input_ids.reshape(NW, n_chunks, CHUNK).astype(jnp.int32)

    mesh = plsc.VectorSubcoreMesh(core_axis_name="c", subcore_axis_name="s")
    k = pl.kernel(
        _gather_kernel,
        out_type=jax.ShapeDtypeStruct((n, D), jnp.float32),
        mesh=mesh,
        scratch_types=[
            pltpu.VMEM((n_chunks, CHUNK), jnp.int32),
            pltpu.VMEM((CHUNK, D), jnp.float32),
            pltpu.VMEM((CHUNK, D), jnp.float32),
            pltpu.SemaphoreType.DMA,
            pltpu.SemaphoreType.DMA,
            pltpu.SemaphoreType.DMA,
            pltpu.SemaphoreType.DMA,
        ],
    )
    out = k(token_emb, idx)
    return out.reshape(B, T, D)
